# baseline (speedup 1.0000x reference)
import functools

import jax
import jax.numpy as jnp
from jax import lax
from jax.experimental import pallas as pl
from jax.experimental.pallas import tpu as pltpu
from jax.experimental.pallas import tpu_sc as plsc

B = 16384
L = 4
VOCAB = 6
EMB = 100
OUT = 2
NSET = 1 << VOCAB

NC = 2
NS = 16
LANES = 16
NW = NC * NS
RPW = B // NW
GROUPS = RPW // LANES

DPAD = 128
NCHUNK = 7
SUBS = NSET // NS

_FMIN = -3.4028235e38


def _sc_body(x_hbm, par_hbm, out_hbm, xv, pv, stagev, lutv, outv, mv, shared, sem):
    cid = lax.axis_index("c")
    sid = lax.axis_index("s")
    wid = sid * NC + cid
    base = wid * RPW

    xcopy = pltpu.async_copy(x_hbm.at[pl.ds(base * L, RPW * L)], xv, sem)
    pltpu.sync_copy(par_hbm, pv)

    lane = lax.iota(jnp.int32, LANES)

    def lut_step(k, stage):
        s = sid * SUBS + k
        szero = jnp.full((LANES,), s == 0)
        sel0 = jnp.full((LANES,), (k & 1) == 1)
        sel1 = jnp.full((LANES,), (k & 2) == 2)
        l0 = jnp.full((LANES,), 0.0, jnp.float32)
        l1 = jnp.full((LANES,), 0.0, jnp.float32)
        for c in range(NCHUNK):
            pool = jnp.full((LANES,), _FMIN, jnp.float32)
            for v in range(2, VOCAB):
                sel = jnp.full((LANES,), ((sid >> (v - 2)) & 1) == 1)
                tv = pv[pl.ds(v * DPAD + c * LANES, LANES)]
                pool = jnp.where(sel, jnp.maximum(pool, tv), pool)
            t0 = pv[pl.ds(c * LANES, LANES)]
            t1 = pv[pl.ds(DPAD + c * LANES, LANES)]
            pool = jnp.where(sel0, jnp.maximum(pool, t0), pool)
            pool = jnp.where(sel1, jnp.maximum(pool, t1), pool)
            pool = jnp.where(szero, 0.0, pool)
            l0 = l0 + pool * pv[pl.ds(VOCAB * DPAD + c * LANES, LANES)]
            l1 = l1 + pool * pv[pl.ds(VOCAB * DPAD + DPAD + c * LANES, LANES)]
        a0 = jnp.full((LANES,), jnp.sum(l0))
        a1 = jnp.full((LANES,), jnp.sum(l1))
        mx = jnp.maximum(a0, a1)
        e0 = jnp.exp(a0 - mx)
        e1 = jnp.exp(a1 - mx)
        tot = e0 + e1
        stage = jnp.where(lane == 2 * k, e0 / tot, stage)
        return jnp.where(lane == 2 * k + 1, e1 / tot, stage)

    stagev[...] = lax.fori_loop(
        0, SUBS, lut_step, jnp.full((LANES,), 0.0, jnp.float32)
    )

    pltpu.sync_copy(stagev, shared.at[pl.ds(sid * LANES, LANES)])

    xcopy.wait()

    one = jnp.full((LANES,), 1, jnp.int32)

    def mask_step(i, _):
        blk = i >> 3
        sub = (i & 7) * LANES
        xoff = blk * (128 * L) + sub
        x0 = xv[pl.ds(xoff, LANES)]
        x1 = xv[pl.ds(xoff + 128, LANES)]
        x2 = xv[pl.ds(xoff + 256, LANES)]
        x3 = xv[pl.ds(xoff + 384, LANES)]
        m = (one << x0) | (one << x1) | (one << x2) | (one << x3)
        loc = (m >> 2) * LANES + (m & 3) * 2
        mv[pl.ds(i * LANES, LANES)] = loc
        return 0

    lax.fori_loop(0, GROUPS, mask_step, 0)

    plsc.subcore_barrier()
    pltpu.sync_copy(shared, lutv)

    def gather_step(i, _):
        blk = i >> 3
        sub = (i & 7) * LANES
        loc = mv[pl.ds(i * LANES, LANES)]
        p0 = plsc.load_gather(lutv, [loc])
        p1 = plsc.load_gather(lutv, [loc + 1])
        ooff = blk * (128 * OUT) + sub
        outv[pl.ds(ooff, LANES)] = p0
        outv[pl.ds(ooff + 128, LANES)] = p1
        return 0

    lax.fori_loop(0, GROUPS, gather_step, 0)
    pltpu.sync_copy(outv, out_hbm.at[pl.ds(base * OUT, RPW * OUT)])


@functools.lru_cache(maxsize=None)
def _make_sc_call():
    return pl.kernel(
        _sc_body,
        out_type=jax.ShapeDtypeStruct((B * OUT,), jnp.float32),
        mesh=plsc.VectorSubcoreMesh(core_axis_name="c", subcore_axis_name="s"),
        compiler_params=pltpu.CompilerParams(needs_layout_passes=False),
        scratch_types=[
            pltpu.VMEM((RPW * L,), jnp.int32),
            pltpu.VMEM(((VOCAB + OUT) * DPAD,), jnp.float32),
            pltpu.VMEM((LANES,), jnp.float32),
            pltpu.VMEM((NS * LANES,), jnp.float32),
            pltpu.VMEM((RPW * OUT,), jnp.float32),
            pltpu.VMEM((RPW,), jnp.int32),
            pltpu.VMEM_SHARED((NS * LANES,), jnp.float32),
            pltpu.SemaphoreType.DMA,
        ],
    )


@jax.jit
def kernel(x, emb_table, W, b):
    x = x.astype(jnp.int32)
    top = jnp.concatenate(
        [emb_table, jnp.ones((VOCAB, 1)), jnp.zeros((VOCAB, DPAD - EMB - 1))],
        axis=1,
        dtype=jnp.float32,
    )
    bot = jnp.concatenate(
        [W.T, b[:, None], jnp.zeros((OUT, DPAD - EMB - 1))],
        axis=1,
        dtype=jnp.float32,
    )
    par = jnp.concatenate([top, bot], axis=0)
    xp = x.reshape(B // 128, 128, L).transpose(0, 2, 1).reshape(B * L)
    out = _make_sc_call()(xp, par.reshape((VOCAB + OUT) * DPAD))
    return out.reshape(B // 128, OUT, 128).transpose(0, 2, 1).reshape(B, OUT)


# speedup vs baseline: 1.0358x; 1.0358x over previous
import functools

import jax
import jax.numpy as jnp
from jax import lax
from jax.experimental import pallas as pl
from jax.experimental.pallas import tpu as pltpu
from jax.experimental.pallas import tpu_sc as plsc

B = 16384
L = 4
VOCAB = 6
EMB = 100
OUT = 2
NSET = 1 << VOCAB

NC = 1
NS = 16
LANES = 16
NW = NC * NS
RPW = B // NW
GROUPS = RPW // LANES

DPAD = 128
NCHUNK = 7
SUBS = NSET // NS

_FMIN = -3.4028235e38


def _sc_body(x_hbm, par_hbm, out_hbm, xv, pv, stagev, lutv, outv, mv, shared, sem):
    cid = lax.axis_index("c")
    sid = lax.axis_index("s")
    wid = sid * NC + cid
    base = wid * RPW

    xcopy = pltpu.async_copy(x_hbm.at[pl.ds(base * L, RPW * L)], xv, sem)
    pltpu.sync_copy(par_hbm, pv)

    lane = lax.iota(jnp.int32, LANES)

    def lut_step(k, stage):
        s = sid * SUBS + k
        szero = jnp.full((LANES,), s == 0)
        sel0 = jnp.full((LANES,), (k & 1) == 1)
        sel1 = jnp.full((LANES,), (k & 2) == 2)
        l0 = jnp.full((LANES,), 0.0, jnp.float32)
        l1 = jnp.full((LANES,), 0.0, jnp.float32)
        for c in range(NCHUNK):
            pool = jnp.full((LANES,), _FMIN, jnp.float32)
            for v in range(2, VOCAB):
                sel = jnp.full((LANES,), ((sid >> (v - 2)) & 1) == 1)
                tv = pv[pl.ds(v * DPAD + c * LANES, LANES)]
                pool = jnp.where(sel, jnp.maximum(pool, tv), pool)
            t0 = pv[pl.ds(c * LANES, LANES)]
            t1 = pv[pl.ds(DPAD + c * LANES, LANES)]
            pool = jnp.where(sel0, jnp.maximum(pool, t0), pool)
            pool = jnp.where(sel1, jnp.maximum(pool, t1), pool)
            pool = jnp.where(szero, 0.0, pool)
            l0 = l0 + pool * pv[pl.ds(VOCAB * DPAD + c * LANES, LANES)]
            l1 = l1 + pool * pv[pl.ds(VOCAB * DPAD + DPAD + c * LANES, LANES)]
        a0 = jnp.full((LANES,), jnp.sum(l0))
        a1 = jnp.full((LANES,), jnp.sum(l1))
        mx = jnp.maximum(a0, a1)
        e0 = jnp.exp(a0 - mx)
        e1 = jnp.exp(a1 - mx)
        tot = e0 + e1
        stage = jnp.where(lane == 2 * k, e0 / tot, stage)
        return jnp.where(lane == 2 * k + 1, e1 / tot, stage)

    stagev[...] = lax.fori_loop(
        0, SUBS, lut_step, jnp.full((LANES,), 0.0, jnp.float32)
    )

    pltpu.sync_copy(stagev, shared.at[pl.ds(sid * LANES, LANES)])

    xcopy.wait()

    one = jnp.full((LANES,), 1, jnp.int32)

    def mask_step(i, _):
        blk = i >> 3
        sub = (i & 7) * LANES
        xoff = blk * (128 * L) + sub
        x0 = xv[pl.ds(xoff, LANES)]
        x1 = xv[pl.ds(xoff + 128, LANES)]
        x2 = xv[pl.ds(xoff + 256, LANES)]
        x3 = xv[pl.ds(xoff + 384, LANES)]
        m = (one << x0) | (one << x1) | (one << x2) | (one << x3)
        loc = (m >> 2) * LANES + (m & 3) * 2
        mv[pl.ds(i * LANES, LANES)] = loc
        return 0

    lax.fori_loop(0, GROUPS, mask_step, 0)

    plsc.subcore_barrier()
    pltpu.sync_copy(shared, lutv)

    def gather_step(i, _):
        blk = i >> 3
        sub = (i & 7) * LANES
        loc = mv[pl.ds(i * LANES, LANES)]
        p0 = plsc.load_gather(lutv, [loc])
        p1 = plsc.load_gather(lutv, [loc + 1])
        ooff = blk * (128 * OUT) + sub
        outv[pl.ds(ooff, LANES)] = p0
        outv[pl.ds(ooff + 128, LANES)] = p1
        return 0

    lax.fori_loop(0, GROUPS, gather_step, 0)
    pltpu.sync_copy(outv, out_hbm.at[pl.ds(base * OUT, RPW * OUT)])


@functools.lru_cache(maxsize=None)
def _make_sc_call():
    return pl.kernel(
        _sc_body,
        out_type=jax.ShapeDtypeStruct((B * OUT,), jnp.float32),
        mesh=plsc.VectorSubcoreMesh(
            core_axis_name="c", subcore_axis_name="s", num_cores=NC
        ),
        compiler_params=pltpu.CompilerParams(needs_layout_passes=False),
        scratch_types=[
            pltpu.VMEM((RPW * L,), jnp.int32),
            pltpu.VMEM(((VOCAB + OUT) * DPAD,), jnp.float32),
            pltpu.VMEM((LANES,), jnp.float32),
            pltpu.VMEM((NS * LANES,), jnp.float32),
            pltpu.VMEM((RPW * OUT,), jnp.float32),
            pltpu.VMEM((RPW,), jnp.int32),
            pltpu.VMEM_SHARED((NS * LANES,), jnp.float32),
            pltpu.SemaphoreType.DMA,
        ],
    )


@jax.jit
def kernel(x, emb_table, W, b):
    x = x.astype(jnp.int32)
    top = jnp.concatenate(
        [emb_table, jnp.ones((VOCAB, 1)), jnp.zeros((VOCAB, DPAD - EMB - 1))],
        axis=1,
        dtype=jnp.float32,
    )
    bot = jnp.concatenate(
        [W.T, b[:, None], jnp.zeros((OUT, DPAD - EMB - 1))],
        axis=1,
        dtype=jnp.float32,
    )
    par = jnp.concatenate([top, bot], axis=0)
    xp = x.reshape(B // 128, 128, L).transpose(0, 2, 1).reshape(B * L)
    out = _make_sc_call()(xp, par.reshape((VOCAB + OUT) * DPAD))
    return out.reshape(B // 128, OUT, 128).transpose(0, 2, 1).reshape(B, OUT)
